# baseline (speedup 1.0000x reference)
import functools

import jax
import jax.numpy as jnp
from jax import lax
from jax.experimental import pallas as pl
from jax.experimental.pallas import tpu as pltpu
from jax.experimental.pallas import tpu_sc as plsc

_NC = 2
_NS = 16
_NW = _NC * _NS
_CH = 128


def _sc_gather_call(ids4d, table):
    nw, nch, _, ch = ids4d.shape
    v, d = table.shape
    per_w = nch * ch
    bs = nw * per_w
    assert nch % 2 == 0

    def body(ids_hbm, table_hbm, out_hbm, idx_v, rows0, rows1, sem0, sem1):
        wid = lax.axis_index("s") * _NC + lax.axis_index("c")
        pltpu.sync_copy(ids_hbm.at[wid], idx_v)

        def gather(j, buf, sem):
            return pltpu.async_copy(table_hbm.at[idx_v.at[j, 0]], buf, sem)

        def drain(j, buf, sem):
            pltpu.make_async_copy(table_hbm.at[idx_v.at[j, 0]], buf, sem).wait()
            pltpu.sync_copy(buf, out_hbm.at[pl.ds(wid * per_w + j * ch, ch)])

        gather(0, rows0, sem0)

        def pair(g, carry):
            j0 = 2 * g
            gather(j0 + 1, rows1, sem1)
            drain(j0, rows0, sem0)

            @pl.when(g + 1 < nch // 2)
            def _():
                gather(j0 + 2, rows0, sem0)

            drain(j0 + 1, rows1, sem1)
            return carry

        lax.fori_loop(0, nch // 2, pair, 0)

    grid_kernel = pl.kernel(
        body,
        out_type=jax.ShapeDtypeStruct((bs, d), jnp.float32),
        mesh=plsc.VectorSubcoreMesh(core_axis_name="c", subcore_axis_name="s"),
        compiler_params=pltpu.CompilerParams(use_tc_tiling_on_sc=False),
        scratch_types=[
            pltpu.VMEM((nch, 1, ch), jnp.int32),
            pltpu.VMEM((ch, d), jnp.float32),
            pltpu.VMEM((ch, d), jnp.float32),
            pltpu.SemaphoreType.DMA,
            pltpu.SemaphoreType.DMA,
        ],
    )
    return grid_kernel(ids4d, table)


def _tc_fused(g_ref, ids_e_ref, ids_o_ref, pos_ref, wt_ref, gamma_ref, beta_ref, out_ref):
    bb, hs = ids_e_ref.shape
    d = wt_ref.shape[0]
    h = out_ref.shape[-1]
    s = 2 * hs
    t2 = bb * hs
    g2 = g_ref[...]

    def half(cols, par_ids):
        m = (par_ids != 0).astype(jnp.float32)[..., None]
        x = (cols.reshape(bb, hs, d) * m).reshape(t2, d)
        y = jnp.dot(x, wt_ref[...], preferred_element_type=jnp.float32)
        return y.reshape(bb, hs, 1, h)

    ye = half(g2[:, :d], ids_e_ref[...])
    yo = half(g2[:, d:], ids_o_ref[...])
    y = jnp.concatenate([ye, yo], axis=2).reshape(bb, s, h)
    y = y + pos_ref[...][None]
    mu = jnp.mean(y, axis=-1, keepdims=True)
    dlt = y - mu
    var = jnp.mean(dlt * dlt, axis=-1, keepdims=True)
    xn = dlt * lax.rsqrt(var + 1e-5)
    out_ref[...] = xn * gamma_ref[...].reshape(1, 1, h) + beta_ref[...].reshape(1, 1, h)


def _tc_call(g2, ids_e, ids_o, pos, wt, gamma2, beta2, bb):
    b, hs = ids_e.shape
    s = 2 * hs
    d, h = wt.shape
    t2 = bb * hs
    grid = (b // bb,)
    return pl.pallas_call(
        _tc_fused,
        grid=grid,
        in_specs=[
            pl.BlockSpec((t2, 2 * d), lambda i: (i, 0)),
            pl.BlockSpec((bb, hs), lambda i: (i, 0)),
            pl.BlockSpec((bb, hs), lambda i: (i, 0)),
            pl.BlockSpec((s, h), lambda i: (0, 0)),
            pl.BlockSpec((d, h), lambda i: (0, 0)),
            pl.BlockSpec((1, h), lambda i: (0, 0)),
            pl.BlockSpec((1, h), lambda i: (0, 0)),
        ],
        out_specs=pl.BlockSpec((bb, s, h), lambda i: (i, 0, 0)),
        out_shape=jax.ShapeDtypeStruct((b, s, h), jnp.float32),
    )(g2, ids_e, ids_o, pos, wt, gamma2, beta2)


def kernel(input_ids, word_table, pos_table, W_proj, gamma, beta):
    b, s = input_ids.shape
    v, d = word_table.shape
    h = pos_table.shape[1]
    ids = input_ids.astype(jnp.int32)
    bs = b * s
    nch = bs // (_NW * _CH)
    ids4d = ids.reshape(_NW, nch, 1, _CH)
    gathered = _sc_gather_call(ids4d, word_table)
    g2 = gathered.reshape(bs // 2, 2 * d)
    out = _tc_call(
        g2,
        ids[:, 0::2],
        ids[:, 1::2],
        pos_table[:s],
        W_proj.T,
        gamma.reshape(1, h),
        beta.reshape(1, h),
        bb=16,
    )
    return out


# speedup vs baseline: 1.2348x; 1.2348x over previous
import functools

import jax
import jax.numpy as jnp
from jax import lax
from jax.experimental import pallas as pl
from jax.experimental.pallas import tpu as pltpu
from jax.experimental.pallas import tpu_sc as plsc

_NC = 2
_NS = 16
_NW = _NC * _NS
_CH = 128


def _sc_gather_call(pids4d, table2):
    nw, nch, _, ch = pids4d.shape
    per_w = nch * ch
    bs = nw * per_w
    assert nch % 2 == 0

    def body(ids_hbm, table_hbm, out_hbm, idx_v, rows0, rows1, sem0, sem1):
        wid = lax.axis_index("s") * _NC + lax.axis_index("c")
        pltpu.sync_copy(ids_hbm.at[wid], idx_v)

        def gather(j, buf, sem):
            return pltpu.async_copy(table_hbm.at[idx_v.at[j, 0]], buf, sem)

        def drain(j, buf, sem):
            pltpu.make_async_copy(table_hbm.at[idx_v.at[j, 0]], buf, sem).wait()
            pltpu.sync_copy(buf, out_hbm.at[pl.ds(wid * per_w + j * ch, ch)])

        gather(0, rows0, sem0)

        def pair(g, carry):
            j0 = 2 * g
            gather(j0 + 1, rows1, sem1)
            drain(j0, rows0, sem0)

            @pl.when(g + 1 < nch // 2)
            def _():
                gather(j0 + 2, rows0, sem0)

            drain(j0 + 1, rows1, sem1)
            return carry

        lax.fori_loop(0, nch // 2, pair, 0)

    grid_kernel = pl.kernel(
        body,
        out_type=jax.ShapeDtypeStruct((bs, 128), jnp.float32),
        mesh=plsc.VectorSubcoreMesh(core_axis_name="c", subcore_axis_name="s"),
        compiler_params=pltpu.CompilerParams(use_tc_tiling_on_sc=True),
        scratch_types=[
            pltpu.VMEM((nch, 1, ch), jnp.int32),
            pltpu.VMEM((ch, 128), jnp.float32),
            pltpu.VMEM((ch, 128), jnp.float32),
            pltpu.SemaphoreType.DMA,
            pltpu.SemaphoreType.DMA,
        ],
    )
    return grid_kernel(pids4d, table2)


def _tc_fused(g_ref, ids_ref, pos_ref, wt_ref, gamma_ref, beta_ref, out_ref):
    bb, s = ids_ref.shape
    d = wt_ref.shape[0]
    h = out_ref.shape[-1]
    t = bb * s
    g3 = g_ref[...].reshape(bb, s, 128)
    ids3 = ids_ref[...][..., None]
    odd = (ids3 & 1) != 0
    half = jnp.where(odd, g3[..., d:], g3[..., :d])
    x = (half * (ids3 != 0).astype(jnp.float32)).reshape(t, d)
    y = jnp.dot(x, wt_ref[...], preferred_element_type=jnp.float32)
    y = y.reshape(bb, s, h) + pos_ref[...][None]
    mu = jnp.mean(y, axis=-1, keepdims=True)
    dlt = y - mu
    var = jnp.mean(dlt * dlt, axis=-1, keepdims=True)
    xn = dlt * lax.rsqrt(var + 1e-5)
    out_ref[...] = xn * gamma_ref[...].reshape(1, 1, h) + beta_ref[...].reshape(1, 1, h)


def _tc_call(g2, ids, pos, wt, gamma2, beta2, bb):
    b, s = ids.shape
    d, h = wt.shape
    t = bb * s
    grid = (b // bb,)
    return pl.pallas_call(
        _tc_fused,
        grid=grid,
        in_specs=[
            pl.BlockSpec((t, 128), lambda i: (i, 0)),
            pl.BlockSpec((bb, s), lambda i: (i, 0)),
            pl.BlockSpec((s, h), lambda i: (0, 0)),
            pl.BlockSpec((d, h), lambda i: (0, 0)),
            pl.BlockSpec((1, h), lambda i: (0, 0)),
            pl.BlockSpec((1, h), lambda i: (0, 0)),
        ],
        out_specs=pl.BlockSpec((bb, s, h), lambda i: (i, 0, 0)),
        out_shape=jax.ShapeDtypeStruct((b, s, h), jnp.float32),
    )(g2, ids, pos, wt, gamma2, beta2)


def kernel(input_ids, word_table, pos_table, W_proj, gamma, beta):
    b, s = input_ids.shape
    v, d = word_table.shape
    h = pos_table.shape[1]
    ids = input_ids.astype(jnp.int32)
    bs = b * s
    nch = bs // (_NW * _CH)
    pids4d = lax.shift_right_logical(ids, 1).reshape(_NW, nch, 1, _CH)
    table2 = word_table.reshape(v // 2, 2 * d)
    gathered = _sc_gather_call(pids4d, table2)
    out = _tc_call(
        gathered,
        ids,
        pos_table[:s],
        W_proj.T,
        gamma.reshape(1, h),
        beta.reshape(1, h),
        bb=16,
    )
    return out


# speedup vs baseline: 1.2355x; 1.0006x over previous
import functools

import jax
import jax.numpy as jnp
from jax import lax
from jax.experimental import pallas as pl
from jax.experimental.pallas import tpu as pltpu
from jax.experimental.pallas import tpu_sc as plsc

_NC = 2
_NS = 16
_NW = _NC * _NS
_CH = 128


def _sc_gather_call(pids4d, table2):
    nw, nch, _, ch = pids4d.shape
    per_w = nch * ch
    bs = nw * per_w
    assert nch % 2 == 0

    nbuf = 5
    assert nch % nbuf == 0

    def body(ids_hbm, table_hbm, out_hbm, idx_v, rows, gs, ss):
        wid = lax.axis_index("s") * _NC + lax.axis_index("c")
        pltpu.sync_copy(ids_hbm.at[wid], idx_v)

        def gather(j, b):
            pltpu.async_copy(table_hbm.at[idx_v.at[j, 0]], rows[b], gs[b])

        def wait_gather(j, b):
            pltpu.make_async_copy(table_hbm.at[idx_v.at[j, 0]], rows[b], gs[b]).wait()

        def out_slice(j):
            return out_hbm.at[pl.ds(wid * per_w + j * ch, ch)]

        def scatter(j, b):
            pltpu.async_copy(rows[b], out_slice(j), ss[b])

        def wait_scatter(j, b):
            pltpu.make_async_copy(rows[b], out_slice(j), ss[b]).wait()

        for b in range(3):
            gather(b, b)

        def group(g, carry):
            for b in range(nbuf):
                j = nbuf * g + b
                wait_gather(j, b)
                scatter(j, b)
                bn = (b + 3) % nbuf

                @pl.when(j + 3 < nch)
                def _():
                    @pl.when(j >= 2)
                    def _():
                        wait_scatter(j - 2, bn)

                    gather(j + 3, bn)

            return carry

        lax.fori_loop(0, nch // nbuf, group, 0)
        for b in range(nbuf):
            wait_scatter(nch - nbuf + b, b)

    grid_kernel = pl.kernel(
        body,
        out_type=jax.ShapeDtypeStruct((bs, 128), jnp.float32),
        mesh=plsc.VectorSubcoreMesh(core_axis_name="c", subcore_axis_name="s"),
        compiler_params=pltpu.CompilerParams(use_tc_tiling_on_sc=True),
        scratch_types=[
            pltpu.VMEM((nch, 1, ch), jnp.int32),
            [pltpu.VMEM((ch, 128), jnp.float32) for _ in range(5)],
            [pltpu.SemaphoreType.DMA for _ in range(5)],
            [pltpu.SemaphoreType.DMA for _ in range(5)],
        ],
    )
    return grid_kernel(pids4d, table2)


def _tc_fused(g_ref, ids_ref, pos_ref, wt_ref, gamma_ref, beta_ref, out_ref):
    bb, s = ids_ref.shape
    d = wt_ref.shape[0]
    h = out_ref.shape[-1]
    t = bb * s
    g3 = g_ref[...].reshape(bb, s, 128)
    ids3 = ids_ref[...][..., None]
    odd = (ids3 & 1) != 0
    half = jnp.where(odd, g3[..., d:], g3[..., :d])
    x = (half * (ids3 != 0).astype(jnp.float32)).reshape(t, d)
    y = jnp.dot(x, wt_ref[...], preferred_element_type=jnp.float32)
    y = y.reshape(bb, s, h) + pos_ref[...][None]
    mu = jnp.mean(y, axis=-1, keepdims=True)
    dlt = y - mu
    var = jnp.mean(dlt * dlt, axis=-1, keepdims=True)
    xn = dlt * lax.rsqrt(var + 1e-5)
    out_ref[...] = xn * gamma_ref[...].reshape(1, 1, h) + beta_ref[...].reshape(1, 1, h)


def _tc_call(g2, ids, pos, wt, gamma2, beta2, bb):
    b, s = ids.shape
    d, h = wt.shape
    t = bb * s
    grid = (b // bb,)
    return pl.pallas_call(
        _tc_fused,
        grid=grid,
        in_specs=[
            pl.BlockSpec((t, 128), lambda i: (i, 0)),
            pl.BlockSpec((bb, s), lambda i: (i, 0)),
            pl.BlockSpec((s, h), lambda i: (0, 0)),
            pl.BlockSpec((d, h), lambda i: (0, 0)),
            pl.BlockSpec((1, h), lambda i: (0, 0)),
            pl.BlockSpec((1, h), lambda i: (0, 0)),
        ],
        out_specs=pl.BlockSpec((bb, s, h), lambda i: (i, 0, 0)),
        out_shape=jax.ShapeDtypeStruct((b, s, h), jnp.float32),
    )(g2, ids, pos, wt, gamma2, beta2)


def kernel(input_ids, word_table, pos_table, W_proj, gamma, beta):
    b, s = input_ids.shape
    v, d = word_table.shape
    h = pos_table.shape[1]
    ids = input_ids.astype(jnp.int32)
    bs = b * s
    nch = bs // (_NW * _CH)
    pids4d = lax.shift_right_logical(ids, 1).reshape(_NW, nch, 1, _CH)
    table2 = word_table.reshape(v // 2, 2 * d)
    gathered = _sc_gather_call(pids4d, table2)
    out = _tc_call(
        gathered,
        ids,
        pos_table[:s],
        W_proj.T,
        gamma.reshape(1, h),
        beta.reshape(1, h),
        bb=16,
    )
    return out


# speedup vs baseline: 1.2356x; 1.0001x over previous
import functools

import jax
import jax.numpy as jnp
from jax import lax
from jax.experimental import pallas as pl
from jax.experimental.pallas import tpu as pltpu
from jax.experimental.pallas import tpu_sc as plsc

_NC = 2
_NS = 16
_NW = _NC * _NS
_CH = 128


def _sc_gather_call(pids4d, table2):
    nw, nch, _, ch = pids4d.shape
    per_w = nch * ch
    bs = nw * per_w
    assert nch % 2 == 0

    nbuf = 5
    assert nch % nbuf == 0

    def body(ids_hbm, table_hbm, out_hbm, idx_v, rows, gs, ss):
        wid = lax.axis_index("s") * _NC + lax.axis_index("c")
        pltpu.sync_copy(ids_hbm.at[wid], idx_v)

        def gather(j, b):
            pltpu.async_copy(table_hbm.at[idx_v.at[j, 0]], rows[b], gs[b])

        def wait_gather(j, b):
            pltpu.make_async_copy(table_hbm.at[idx_v.at[j, 0]], rows[b], gs[b]).wait()

        def out_slice(j):
            return out_hbm.at[pl.ds(wid * per_w + j * ch, ch)]

        def scatter(j, b):
            pltpu.async_copy(rows[b], out_slice(j), ss[b])

        def wait_scatter(j, b):
            pltpu.make_async_copy(rows[b], out_slice(j), ss[b]).wait()

        for b in range(3):
            gather(b, b)

        def group(g, carry):
            for b in range(nbuf):
                j = nbuf * g + b
                wait_gather(j, b)
                scatter(j, b)
                bn = (b + 3) % nbuf

                @pl.when(j + 3 < nch)
                def _():
                    @pl.when(j >= 2)
                    def _():
                        wait_scatter(j - 2, bn)

                    gather(j + 3, bn)

            return carry

        lax.fori_loop(0, nch // nbuf, group, 0)
        for b in range(nbuf):
            wait_scatter(nch - nbuf + b, b)

    grid_kernel = pl.kernel(
        body,
        out_type=jax.ShapeDtypeStruct((bs, 128), jnp.float32),
        mesh=plsc.VectorSubcoreMesh(core_axis_name="c", subcore_axis_name="s"),
        compiler_params=pltpu.CompilerParams(use_tc_tiling_on_sc=False),
        scratch_types=[
            pltpu.VMEM((nch, 1, ch), jnp.int32),
            [pltpu.VMEM((ch, 128), jnp.float32) for _ in range(5)],
            [pltpu.SemaphoreType.DMA for _ in range(5)],
            [pltpu.SemaphoreType.DMA for _ in range(5)],
        ],
    )
    return grid_kernel(pids4d, table2)


def _tc_fused(g_ref, ids_ref, pos_ref, wt_ref, gamma_ref, beta_ref, out_ref):
    bb, s = ids_ref.shape
    d = wt_ref.shape[0]
    h = out_ref.shape[-1]
    t = bb * s
    g3 = g_ref[...].reshape(bb, s, 128)
    ids3 = ids_ref[...][..., None]
    odd = (ids3 & 1) != 0
    half = jnp.where(odd, g3[..., d:], g3[..., :d])
    x = (half * (ids3 != 0).astype(jnp.float32)).reshape(t, d)
    y = jnp.dot(x, wt_ref[...], preferred_element_type=jnp.float32)
    y = y.reshape(bb, s, h) + pos_ref[...][None]
    mu = jnp.mean(y, axis=-1, keepdims=True)
    dlt = y - mu
    var = jnp.mean(dlt * dlt, axis=-1, keepdims=True)
    xn = dlt * lax.rsqrt(var + 1e-5)
    out_ref[...] = xn * gamma_ref[...].reshape(1, 1, h) + beta_ref[...].reshape(1, 1, h)


def _tc_call(g2, ids, pos, wt, gamma2, beta2, bb):
    b, s = ids.shape
    d, h = wt.shape
    t = bb * s
    grid = (b // bb,)
    return pl.pallas_call(
        _tc_fused,
        grid=grid,
        in_specs=[
            pl.BlockSpec((t, 128), lambda i: (i, 0)),
            pl.BlockSpec((bb, s), lambda i: (i, 0)),
            pl.BlockSpec((s, h), lambda i: (0, 0)),
            pl.BlockSpec((d, h), lambda i: (0, 0)),
            pl.BlockSpec((1, h), lambda i: (0, 0)),
            pl.BlockSpec((1, h), lambda i: (0, 0)),
        ],
        out_specs=pl.BlockSpec((bb, s, h), lambda i: (i, 0, 0)),
        out_shape=jax.ShapeDtypeStruct((b, s, h), jnp.float32),
    )(g2, ids, pos, wt, gamma2, beta2)


def kernel(input_ids, word_table, pos_table, W_proj, gamma, beta):
    b, s = input_ids.shape
    v, d = word_table.shape
    h = pos_table.shape[1]
    ids = input_ids.astype(jnp.int32)
    bs = b * s
    nch = bs // (_NW * _CH)
    pids4d = lax.shift_right_logical(ids, 1).reshape(_NW, nch, 1, _CH)
    table2 = word_table.reshape(v // 2, 2 * d)
    gathered = _sc_gather_call(pids4d, table2)
    out = _tc_call(
        gathered,
        ids,
        pos_table[:s],
        W_proj.T,
        gamma.reshape(1, h),
        beta.reshape(1, h),
        bb=16,
    )
    return out


# speedup vs baseline: 1.2371x; 1.0013x over previous
import jax
import jax.numpy as jnp
from jax import lax
from jax.experimental import pallas as pl
from jax.experimental.pallas import tpu as pltpu
from jax.experimental.pallas import tpu_sc as plsc

_NC = 2
_NS = 16
_NW = _NC * _NS
_CH = 128


def _sc_gather_call(pids4d, table2):
    nw, nch, _, ch = pids4d.shape
    per_w = nch * ch
    bs = nw * per_w
    assert nch % 2 == 0

    nbuf = 5
    assert nch % nbuf == 0

    def body(ids_hbm, table_hbm, out_hbm, idx_v, rows, gs, ss):
        wid = lax.axis_index("s") * _NC + lax.axis_index("c")
        pltpu.sync_copy(ids_hbm.at[wid], idx_v)

        def gather(j, b):
            pltpu.async_copy(table_hbm.at[idx_v.at[j, 0]], rows[b], gs[b])

        def wait_gather(j, b):
            pltpu.make_async_copy(table_hbm.at[idx_v.at[j, 0]], rows[b], gs[b]).wait()

        def out_slice(j):
            return out_hbm.at[pl.ds(wid * per_w + j * ch, ch)]

        def scatter(j, b):
            pltpu.async_copy(rows[b], out_slice(j), ss[b])

        def wait_scatter(j, b):
            pltpu.make_async_copy(rows[b], out_slice(j), ss[b]).wait()

        for b in range(3):
            gather(b, b)

        def group(g, carry):
            for b in range(nbuf):
                j = nbuf * g + b
                wait_gather(j, b)
                scatter(j, b)
                bn = (b + 3) % nbuf

                @pl.when(j + 3 < nch)
                def _():
                    @pl.when(j >= 2)
                    def _():
                        wait_scatter(j - 2, bn)

                    gather(j + 3, bn)

            return carry

        lax.fori_loop(0, nch // nbuf, group, 0)
        for b in range(nbuf):
            wait_scatter(nch - nbuf + b, b)

    grid_kernel = pl.kernel(
        body,
        out_type=jax.ShapeDtypeStruct((bs, 128), jnp.float32),
        mesh=plsc.VectorSubcoreMesh(core_axis_name="c", subcore_axis_name="s"),
        compiler_params=pltpu.CompilerParams(use_tc_tiling_on_sc=False),
        scratch_types=[
            pltpu.VMEM((nch, 1, ch), jnp.int32),
            [pltpu.VMEM((ch, 128), jnp.float32) for _ in range(5)],
            [pltpu.SemaphoreType.DMA for _ in range(5)],
            [pltpu.SemaphoreType.DMA for _ in range(5)],
        ],
    )
    return grid_kernel(pids4d, table2)


def _tc_fused(g_ref, ids_ref, pos_ref, wt_ref, gamma_ref, beta_ref, out_ref):
    bb, s = ids_ref.shape
    d = wt_ref.shape[0]
    h = out_ref.shape[-1]
    t = bb * s
    g3 = g_ref[...].reshape(bb, s, 128)
    ids3 = ids_ref[...][..., None]
    odd = (ids3 & 1) != 0
    half = jnp.where(odd, g3[..., d:], g3[..., :d])
    x = (half * (ids3 != 0).astype(jnp.float32)).reshape(t, d)
    y = jnp.dot(x, wt_ref[...], preferred_element_type=jnp.float32)
    y = y.reshape(bb, s, h) + pos_ref[...][None]
    mu = jnp.mean(y, axis=-1, keepdims=True)
    dlt = y - mu
    var = jnp.mean(dlt * dlt, axis=-1, keepdims=True)
    xn = dlt * lax.rsqrt(var + 1e-5)
    out_ref[...] = xn * gamma_ref[...].reshape(1, 1, h) + beta_ref[...].reshape(1, 1, h)


def _tc_call(g2, ids, pos, wt, gamma2, beta2, bb):
    b, s = ids.shape
    d, h = wt.shape
    t = bb * s
    grid = (b // bb,)
    return pl.pallas_call(
        _tc_fused,
        grid=grid,
        in_specs=[
            pl.BlockSpec((t, 128), lambda i: (i, 0)),
            pl.BlockSpec((bb, s), lambda i: (i, 0)),
            pl.BlockSpec((s, h), lambda i: (0, 0)),
            pl.BlockSpec((d, h), lambda i: (0, 0)),
            pl.BlockSpec((1, h), lambda i: (0, 0)),
            pl.BlockSpec((1, h), lambda i: (0, 0)),
        ],
        out_specs=pl.BlockSpec((bb, s, h), lambda i: (i, 0, 0)),
        out_shape=jax.ShapeDtypeStruct((b, s, h), jnp.float32),
    )(g2, ids, pos, wt, gamma2, beta2)


def kernel(input_ids, word_table, pos_table, W_proj, gamma, beta):
    b, s = input_ids.shape
    v, d = word_table.shape
    h = pos_table.shape[1]
    ids = input_ids.astype(jnp.int32)
    bs = b * s
    nch = bs // (_NW * _CH)
    pids4d = lax.shift_right_logical(ids, 1).reshape(_NW, nch, 1, _CH)
    table2 = word_table.reshape(v // 2, 2 * d)
    gathered = _sc_gather_call(pids4d, table2)
    out = _tc_call(
        gathered,
        ids,
        pos_table[:s],
        W_proj.T,
        gamma.reshape(1, h),
        beta.reshape(1, h),
        bb=16,
    )
    return out
